# Initial kernel scaffold; baseline (speedup 1.0000x reference)
import jax
import jax.numpy as jnp
from jax.experimental import pallas as pl


def kernel(x, edge_index, W1, a_src1, a_dst1, b1, W2, a_src2, a_dst2, b2, W_out, b_out):
    raise NotImplementedError("write your pallas kernel here")


# speedup vs baseline: 47.1804x; 47.1804x over previous
import functools

import jax
import jax.numpy as jnp
from jax import lax
from jax.experimental import pallas as pl
from jax.experimental.pallas import tpu as pltpu
from jax.experimental.pallas import tpu_sc as plsc

_N = 10000
_D = 128
_H = 8
_HID = 16
_C = 40
_NP = 10240
_NC = 2
_NS = 16
_NW = _NC * _NS
_B = 128
_NB = 81
_EPAD = _NW * _NB * _B
_BLK = 256


def _mm_attn_body(x_ref, w_ref, am_ref, h_ref, aa_ref):
    h = jnp.dot(x_ref[...], w_ref[...], preferred_element_type=jnp.float32)
    h_ref[...] = h
    aa_ref[...] = jnp.dot(h, am_ref[...], preferred_element_type=jnp.float32)


def _mm_attn(xp, w, am):
    return pl.pallas_call(
        _mm_attn_body,
        grid=(_NP // _BLK,),
        in_specs=[
            pl.BlockSpec((_BLK, _D), lambda i: (i, 0)),
            pl.BlockSpec((_D, _D), lambda i: (0, 0)),
            pl.BlockSpec((_D, 16), lambda i: (0, 0)),
        ],
        out_specs=[
            pl.BlockSpec((_BLK, _D), lambda i: (i, 0)),
            pl.BlockSpec((_BLK, 16), lambda i: (i, 0)),
        ],
        out_shape=[
            jax.ShapeDtypeStruct((_NP, _D), jnp.float32),
            jax.ShapeDtypeStruct((_NP, 16), jnp.float32),
        ],
    )(xp, w, am)


def _bound_body(aa_ref, a_ref):
    m = jnp.max(aa_ref[...], axis=0, keepdims=True)
    s = m[:, :_H] + m[:, _H:]
    a = jnp.maximum(s, 0.2 * s)
    a_ref[...] = jnp.concatenate([a, a], axis=1)


def _bound(aa):
    return pl.pallas_call(
        _bound_body,
        out_shape=jax.ShapeDtypeStruct((1, 16), jnp.float32),
    )(aa)


def _fin_body(np_ref, dp_ref, exp_ref, b_ref, w_ref, am_ref, h_ref, aa_ref):
    num = np_ref[0] + np_ref[1]
    den = dp_ref[0] + dp_ref[1]
    dex = jnp.dot(den, exp_ref[...], preferred_element_type=jnp.float32) + 1e-16
    h1 = jnp.maximum(num / dex + b_ref[...], 0.0)
    h2 = jnp.dot(h1, w_ref[...], preferred_element_type=jnp.float32)
    h_ref[...] = h2
    aa_ref[...] = jnp.dot(h2, am_ref[...], preferred_element_type=jnp.float32)


def _fin(nump, denp, expand, b, w, am):
    return pl.pallas_call(
        _fin_body,
        grid=(_NP // _BLK,),
        in_specs=[
            pl.BlockSpec((2, _BLK, _D), lambda i: (0, i, 0)),
            pl.BlockSpec((2, _BLK, 16), lambda i: (0, i, 0)),
            pl.BlockSpec((16, _D), lambda i: (0, 0)),
            pl.BlockSpec((1, _D), lambda i: (0, 0)),
            pl.BlockSpec((_D, _D), lambda i: (0, 0)),
            pl.BlockSpec((_D, 16), lambda i: (0, 0)),
        ],
        out_specs=[
            pl.BlockSpec((_BLK, _D), lambda i: (i, 0)),
            pl.BlockSpec((_BLK, 16), lambda i: (i, 0)),
        ],
        out_shape=[
            jax.ShapeDtypeStruct((_NP, _D), jnp.float32),
            jax.ShapeDtypeStruct((_NP, 16), jnp.float32),
        ],
    )(nump, denp, expand, b, w, am)


def _out_body(np_ref, dp_ref, exp_ref, b_ref, w_ref, bo_ref, o_ref):
    num = np_ref[0] + np_ref[1]
    den = dp_ref[0] + dp_ref[1]
    dex = jnp.dot(den, exp_ref[...], preferred_element_type=jnp.float32) + 1e-16
    h2 = jnp.maximum(num / dex + b_ref[...], 0.0)
    o_ref[...] = jnp.dot(h2, w_ref[...], preferred_element_type=jnp.float32) + bo_ref[...]


def _out(nump, denp, expand, b, w, bo):
    return pl.pallas_call(
        _out_body,
        grid=(_NP // _BLK,),
        in_specs=[
            pl.BlockSpec((2, _BLK, _D), lambda i: (0, i, 0)),
            pl.BlockSpec((2, _BLK, 16), lambda i: (0, i, 0)),
            pl.BlockSpec((16, _D), lambda i: (0, 0)),
            pl.BlockSpec((1, _D), lambda i: (0, 0)),
            pl.BlockSpec((_D, _D), lambda i: (0, 0)),
            pl.BlockSpec((1, _D), lambda i: (0, 0)),
        ],
        out_specs=pl.BlockSpec((_BLK, _D), lambda i: (i, 0)),
        out_shape=jax.ShapeDtypeStruct((_NP, _D), jnp.float32),
    )(nump, denp, expand, b, w, bo)


_SC_PARAMS = pltpu.CompilerParams(
    needs_layout_passes=False, use_tc_tiling_on_sc=False)
_STRIPE = _NP // _NS
_NZ = _STRIPE // _B


def _den_body(aa1_hbm, aa2_hbm, a_hbm, src_hbm, dst_hbm,
              denp_hbm, ex_hbm,
              src_v, dst_v, ag1_v, ag2_v, ex_v, a_v, den_sp,
              sem1, sem2):
    cid = lax.axis_index("c")
    sid = lax.axis_index("s")
    wid = sid * _NC + cid

    pltpu.sync_copy(src_hbm.at[wid], src_v)
    pltpu.sync_copy(dst_hbm.at[wid], dst_v)
    pltpu.sync_copy(a_hbm, a_v)
    areg = a_v[0, :]

    def zrow(e, c):
        ag1_v[e, :] = jnp.zeros((16,), jnp.float32)
        return c
    lax.fori_loop(0, _B, zrow, 0)

    def zacc(t, c):
        r = sid * _STRIPE + t * _B
        pltpu.sync_copy(ag1_v, den_sp.at[pl.ds(r, _B)])
        return c
    lax.fori_loop(0, _NZ, zacc, 0)
    plsc.subcore_barrier()

    def blk(b, c):
        sidx = src_v.at[b]
        didx = dst_v.at[b]
        c2 = pltpu.async_copy(aa1_hbm.at[sidx], ag1_v, sem1)
        c3 = pltpu.async_copy(aa2_hbm.at[didx], ag2_v, sem2)
        c2.wait()
        c3.wait()

        def exloop(e, cc):
            s = ag1_v[e, :] + ag2_v[e, :]
            al = jnp.maximum(s, 0.2 * s)
            ex_v[e, :] = jnp.exp(al - areg)
            return cc
        lax.fori_loop(0, _B, exloop, 0)
        pltpu.sync_copy(ex_v, den_sp.at[didx], add=True)
        pltpu.sync_copy(ex_v, ex_hbm.at[wid].at[b])
        return c
    lax.fori_loop(0, _NB, blk, 0)
    plsc.subcore_barrier()

    def dump(t, c):
        r = sid * _STRIPE + t * _B
        pltpu.sync_copy(den_sp.at[pl.ds(r, _B)], ex_v)
        pltpu.sync_copy(ex_v, denp_hbm.at[cid, pl.ds(r, _B)])
        return c
    lax.fori_loop(0, _NZ, dump, 0)


_den_call = functools.partial(
    pl.kernel,
    out_type=[
        jax.ShapeDtypeStruct((_NC, _NP, 16), jnp.float32),
        jax.ShapeDtypeStruct((_NW, _NB, _B, 16), jnp.float32),
    ],
    mesh=plsc.VectorSubcoreMesh(core_axis_name="c", subcore_axis_name="s"),
    compiler_params=_SC_PARAMS,
    scratch_types=[
        pltpu.VMEM((_NB, _B), jnp.int32),
        pltpu.VMEM((_NB, _B), jnp.int32),
        pltpu.VMEM((_B, 16), jnp.float32),
        pltpu.VMEM((_B, 16), jnp.float32),
        pltpu.VMEM((_B, 16), jnp.float32),
        pltpu.VMEM((1, 16), jnp.float32),
        pltpu.VMEM_SHARED((_NP, 16), jnp.float32),
        pltpu.SemaphoreType.DMA,
        pltpu.SemaphoreType.DMA,
    ],
)(_den_body)


def _num_body(h_hbm, ex_hbm, src_hbm, dst_hbm,
              nump_hbm,
              src_v, dst_v, rows_v, ex_v, num_sp,
              sem1):
    cid = lax.axis_index("c")
    sid = lax.axis_index("s")
    wid = sid * _NC + cid

    pltpu.sync_copy(src_hbm.at[wid], src_v)
    pltpu.sync_copy(dst_hbm.at[wid], dst_v)

    def zrow(e, c):
        for k in range(_H):
            rows_v[e, pl.ds(16 * k, 16)] = jnp.zeros((16,), jnp.float32)
        return c
    lax.fori_loop(0, _B, zrow, 0)

    def zacc(t, c):
        r = sid * _STRIPE + t * _B
        pltpu.sync_copy(rows_v, num_sp.at[pl.ds(r, _B)])
        return c
    lax.fori_loop(0, _NZ, zacc, 0)
    plsc.subcore_barrier()

    def blk(b, c):
        sidx = src_v.at[b]
        didx = dst_v.at[b]
        c1 = pltpu.async_copy(h_hbm.at[sidx], rows_v, sem1)
        pltpu.sync_copy(ex_hbm.at[wid].at[b], ex_v)
        c1.wait()

        def scloop(e, cc):
            e16 = jnp.full((16,), e, jnp.int32)
            for hh in range(_H):
                h16 = jnp.full((16,), hh, jnp.int32)
                cef = plsc.load_gather(ex_v, [e16, h16])
                rows_v[e, pl.ds(16 * hh, 16)] = rows_v[e, pl.ds(16 * hh, 16)] * cef
            return cc
        lax.fori_loop(0, _B, scloop, 0)
        pltpu.sync_copy(rows_v, num_sp.at[didx], add=True)
        return c
    lax.fori_loop(0, _NB, blk, 0)
    plsc.subcore_barrier()

    def dump(t, c):
        r = sid * _STRIPE + t * _B
        pltpu.sync_copy(num_sp.at[pl.ds(r, _B)], rows_v)
        pltpu.sync_copy(rows_v, nump_hbm.at[cid, pl.ds(r, _B)])
        return c
    lax.fori_loop(0, _NZ, dump, 0)


_num_call = functools.partial(
    pl.kernel,
    out_type=jax.ShapeDtypeStruct((_NC, _NP, _D), jnp.float32),
    mesh=plsc.VectorSubcoreMesh(core_axis_name="c", subcore_axis_name="s"),
    compiler_params=_SC_PARAMS,
    scratch_types=[
        pltpu.VMEM((_NB, _B), jnp.int32),
        pltpu.VMEM((_NB, _B), jnp.int32),
        pltpu.VMEM((_B, _D), jnp.float32),
        pltpu.VMEM((_B, 16), jnp.float32),
        pltpu.VMEM_SHARED((_NP, _D), jnp.float32),
        pltpu.SemaphoreType.DMA,
    ],
)(_num_body)


def _edge(h, aa, aab, a, src, dst):
    denp, ex = _den_call(aa, aab, a, src, dst)
    nump = _num_call(h, ex, src, dst)
    return nump, denp


def kernel(x, edge_index, W1, a_src1, a_dst1, b1, W2, a_src2, a_dst2, b2,
           W_out, b_out):
    xp = jnp.zeros((_NP, _D), jnp.float32).at[:_N].set(x)
    loop = jnp.arange(_N, dtype=jnp.int32)
    src = jnp.concatenate([edge_index[0], loop])
    dst = jnp.concatenate([edge_index[1], loop])
    pad = _EPAD - src.shape[0]
    padv = jnp.full((pad,), _N, jnp.int32)
    src = jnp.concatenate([src, padv]).reshape(_NW, _NB, _B)
    dst = jnp.concatenate([dst, padv]).reshape(_NW, _NB, _B)

    M = jnp.repeat(jnp.eye(_H, dtype=jnp.float32), _HID, axis=0)
    am1 = jnp.concatenate(
        [M * a_src1.reshape(-1, 1), M * a_dst1.reshape(-1, 1)], axis=1)
    am2 = jnp.concatenate(
        [M * a_src2.reshape(-1, 1), M * a_dst2.reshape(-1, 1)], axis=1)
    expand = jnp.concatenate([M.T, jnp.zeros((_H, _D), jnp.float32)], axis=0)
    wout = jnp.zeros((_D, _D), jnp.float32).at[:, :_C].set(W_out)
    bout = jnp.zeros((1, _D), jnp.float32).at[0, :_C].set(b_out)

    h1p, aa1 = _mm_attn(xp, W1, am1)
    a1 = _bound(aa1)
    aa1b = jnp.roll(aa1, _H, axis=1)
    num1, den1 = _edge(h1p, aa1, aa1b, a1, src, dst)

    h2p, aa2 = _fin(num1, den1, expand, b1.reshape(1, -1), W2, am2)
    a2 = _bound(aa2)
    aa2b = jnp.roll(aa2, _H, axis=1)
    num2, den2 = _edge(h2p, aa2, aa2b, a2, src, dst)

    outp = _out(num2, den2, expand, b2.reshape(1, -1), wout, bout)
    return outp[:_N, :_C]


# speedup vs baseline: 82.0472x; 1.7390x over previous
import functools

import jax
import jax.numpy as jnp
from jax import lax
from jax.experimental import pallas as pl
from jax.experimental.pallas import tpu as pltpu
from jax.experimental.pallas import tpu_sc as plsc

_N = 10000
_D = 128
_H = 8
_HID = 16
_C = 40
_NP = 10240
_NC = 2
_NS = 16
_NW = _NC * _NS
_B = 128
_NB = 81
_EPAD = _NW * _NB * _B
_BLK = 256


def _accum_bound(aa, a_ref, i):
    bm = jnp.max(aa, axis=0, keepdims=True)
    @pl.when(i == 0)
    def _():
        a_ref[...] = jnp.full((1, 16), -jnp.inf, jnp.float32)
    a_ref[...] = jnp.maximum(a_ref[...], bm)

    @pl.when(i == _NP // _BLK - 1)
    def _():
        m = a_ref[...]
        s = m[:, :_H] + m[:, _H:]
        a = jnp.maximum(s, 0.2 * s)
        a_ref[...] = jnp.concatenate([a, a], axis=1)


def _mm_attn_body(x_ref, w_ref, am_ref, h_ref, aa_ref, a_ref):
    h = jnp.dot(x_ref[...], w_ref[...], preferred_element_type=jnp.float32)
    h_ref[...] = h
    aa = jnp.dot(h, am_ref[...], preferred_element_type=jnp.float32)
    aa_ref[...] = aa
    _accum_bound(aa, a_ref, pl.program_id(0))


def _mm_attn(xp, w, am):
    return pl.pallas_call(
        _mm_attn_body,
        grid=(_NP // _BLK,),
        in_specs=[
            pl.BlockSpec((_BLK, _D), lambda i: (i, 0)),
            pl.BlockSpec((_D, _D), lambda i: (0, 0)),
            pl.BlockSpec((_D, 16), lambda i: (0, 0)),
        ],
        out_specs=[
            pl.BlockSpec((_BLK, _D), lambda i: (i, 0)),
            pl.BlockSpec((_BLK, 16), lambda i: (i, 0)),
            pl.BlockSpec((1, 16), lambda i: (0, 0)),
        ],
        out_shape=[
            jax.ShapeDtypeStruct((_NP, _D), jnp.float32),
            jax.ShapeDtypeStruct((_NP, 16), jnp.float32),
            jax.ShapeDtypeStruct((1, 16), jnp.float32),
        ],
    )(xp, w, am)


def _fin_body(np_ref, dp_ref, exp_ref, b_ref, w_ref, am_ref, h_ref, aa_ref,
              a_ref):
    num = np_ref[0] + np_ref[1]
    den = dp_ref[0] + dp_ref[1]
    dex = jnp.dot(den, exp_ref[...], preferred_element_type=jnp.float32) + 1e-16
    h1 = jnp.maximum(num / dex + b_ref[...], 0.0)
    h2 = jnp.dot(h1, w_ref[...], preferred_element_type=jnp.float32)
    h_ref[...] = h2
    aa = jnp.dot(h2, am_ref[...], preferred_element_type=jnp.float32)
    aa_ref[...] = aa
    _accum_bound(aa, a_ref, pl.program_id(0))


def _fin(nump, denp, expand, b, w, am):
    return pl.pallas_call(
        _fin_body,
        grid=(_NP // _BLK,),
        in_specs=[
            pl.BlockSpec((2, _BLK, _D), lambda i: (0, i, 0)),
            pl.BlockSpec((2, _BLK, _H), lambda i: (0, i, 0)),
            pl.BlockSpec((_H, _D), lambda i: (0, 0)),
            pl.BlockSpec((1, _D), lambda i: (0, 0)),
            pl.BlockSpec((_D, _D), lambda i: (0, 0)),
            pl.BlockSpec((_D, 16), lambda i: (0, 0)),
        ],
        out_specs=[
            pl.BlockSpec((_BLK, _D), lambda i: (i, 0)),
            pl.BlockSpec((_BLK, 16), lambda i: (i, 0)),
            pl.BlockSpec((1, 16), lambda i: (0, 0)),
        ],
        out_shape=[
            jax.ShapeDtypeStruct((_NP, _D), jnp.float32),
            jax.ShapeDtypeStruct((_NP, 16), jnp.float32),
            jax.ShapeDtypeStruct((1, 16), jnp.float32),
        ],
    )(nump, denp, expand, b, w, am)


def _out_body(np_ref, dp_ref, exp_ref, b_ref, w_ref, bo_ref, o_ref):
    num = np_ref[0] + np_ref[1]
    den = dp_ref[0] + dp_ref[1]
    dex = jnp.dot(den, exp_ref[...], preferred_element_type=jnp.float32) + 1e-16
    h2 = jnp.maximum(num / dex + b_ref[...], 0.0)
    o_ref[...] = jnp.dot(h2, w_ref[...], preferred_element_type=jnp.float32) + bo_ref[...]


def _out(nump, denp, expand, b, w, bo):
    return pl.pallas_call(
        _out_body,
        grid=(_NP // _BLK,),
        in_specs=[
            pl.BlockSpec((2, _BLK, _D), lambda i: (0, i, 0)),
            pl.BlockSpec((2, _BLK, _H), lambda i: (0, i, 0)),
            pl.BlockSpec((_H, _D), lambda i: (0, 0)),
            pl.BlockSpec((1, _D), lambda i: (0, 0)),
            pl.BlockSpec((_D, _D), lambda i: (0, 0)),
            pl.BlockSpec((1, _D), lambda i: (0, 0)),
        ],
        out_specs=pl.BlockSpec((_BLK, _D), lambda i: (i, 0)),
        out_shape=jax.ShapeDtypeStruct((_NP, _D), jnp.float32),
    )(nump, denp, expand, b, w, bo)


_SC_PARAMS = pltpu.CompilerParams(
    needs_layout_passes=False, use_tc_tiling_on_sc=False)
_STRIPE = _NP // _NS
_NZ = _STRIPE // _B


def _edge_body(h_hbm, aa1_hbm, aa2_hbm, a_hbm, src_hbm, dst_hbm,
               nump_hbm, denp_hbm,
               src_v, dst_v, rows_v, ag1_v, ag2_v, ex_v, a_v,
               num_sp, den_sp,
               sem1, sem2, sem3):
    cid = lax.axis_index("c")
    sid = lax.axis_index("s")
    wid = sid * _NC + cid

    pltpu.sync_copy(src_hbm.at[wid], src_v)
    pltpu.sync_copy(dst_hbm.at[wid], dst_v)
    pltpu.sync_copy(a_hbm, a_v)
    areg = a_v[0, :]

    iot = lax.iota(jnp.int32, 16)
    rowoff = jnp.where(iot < _H, 0, 1)
    colv = jnp.bitwise_and(iot, _H - 1)

    def zrow(e, c):
        for k in range(_H):
            rows_v[e, pl.ds(16 * k, 16)] = jnp.zeros((16,), jnp.float32)
        return c
    lax.fori_loop(0, _B, zrow, 0)

    def zex(e2, c):
        plsc.store_scatter(
            ex_v, [jnp.full((16,), 2 * e2, jnp.int32) + rowoff, colv],
            jnp.zeros((16,), jnp.float32))
        return c
    lax.fori_loop(0, _B // 2, zex, 0)

    def zacc(t, c):
        r = sid * _STRIPE + t * _B
        pltpu.sync_copy(rows_v, num_sp.at[pl.ds(r, _B)])
        pltpu.sync_copy(ex_v, den_sp.at[pl.ds(r, _B)])
        return c
    lax.fori_loop(0, _NZ, zacc, 0)
    plsc.subcore_barrier()

    def blk(b, c):
        sidx = src_v.at[b]
        didx = dst_v.at[b]
        c1 = pltpu.async_copy(h_hbm.at[sidx], rows_v, sem1)
        c2 = pltpu.async_copy(aa1_hbm.at[sidx], ag1_v, sem2)
        c3 = pltpu.async_copy(aa2_hbm.at[didx], ag2_v, sem3)
        c2.wait()
        c3.wait()

        c1.wait()

        def exsc(e2, cc):
            e = 2 * e2
            rowv = jnp.full((16,), e, jnp.int32) + rowoff
            a1 = plsc.load_gather(ag1_v, [rowv, colv])
            a2 = plsc.load_gather(ag2_v, [rowv, colv])
            s = a1 + a2
            al = jnp.maximum(s, 0.2 * s)
            ex = jnp.exp(al - areg)
            plsc.store_scatter(ex_v, [rowv, colv], ex)
            for hh in range(_H):
                cef = jnp.full((16,), ex[hh], jnp.float32)
                rows_v[e, pl.ds(16 * hh, 16)] = rows_v[e, pl.ds(16 * hh, 16)] * cef
            for hh in range(_H):
                cef = jnp.full((16,), ex[_H + hh], jnp.float32)
                rows_v[e + 1, pl.ds(16 * hh, 16)] = rows_v[e + 1, pl.ds(16 * hh, 16)] * cef
            return cc
        lax.fori_loop(0, _B // 2, exsc, 0)
        pltpu.sync_copy(ex_v, den_sp.at[didx], add=True)
        pltpu.sync_copy(rows_v, num_sp.at[didx], add=True)
        return c
    lax.fori_loop(0, _NB, blk, 0)
    plsc.subcore_barrier()

    def dump(t, c):
        r = sid * _STRIPE + t * _B
        pltpu.sync_copy(num_sp.at[pl.ds(r, _B)], rows_v)
        pltpu.sync_copy(rows_v, nump_hbm.at[cid, pl.ds(r, _B)])
        pltpu.sync_copy(den_sp.at[pl.ds(r, _B)], ex_v)
        pltpu.sync_copy(ex_v, denp_hbm.at[cid, pl.ds(r, _B)])
        return c
    lax.fori_loop(0, _NZ, dump, 0)


_edge_call = functools.partial(
    pl.kernel,
    out_type=[
        jax.ShapeDtypeStruct((_NC, _NP, _D), jnp.float32),
        jax.ShapeDtypeStruct((_NC, _NP, _H), jnp.float32),
    ],
    mesh=plsc.VectorSubcoreMesh(core_axis_name="c", subcore_axis_name="s"),
    compiler_params=_SC_PARAMS,
    scratch_types=[
        pltpu.VMEM((_NB, _B), jnp.int32),
        pltpu.VMEM((_NB, _B), jnp.int32),
        pltpu.VMEM((_B, _D), jnp.float32),
        pltpu.VMEM((_B, _H), jnp.float32),
        pltpu.VMEM((_B, _H), jnp.float32),
        pltpu.VMEM((_B, _H), jnp.float32),
        pltpu.VMEM((1, 16), jnp.float32),
        pltpu.VMEM_SHARED((_NP, _D), jnp.float32),
        pltpu.VMEM_SHARED((_NP, _H), jnp.float32),
        pltpu.SemaphoreType.DMA,
        pltpu.SemaphoreType.DMA,
        pltpu.SemaphoreType.DMA,
    ],
)(_edge_body)


def _edge(h, aas, aad, a, src, dst):
    return _edge_call(h, aas, aad, a, src, dst)


def kernel(x, edge_index, W1, a_src1, a_dst1, b1, W2, a_src2, a_dst2, b2,
           W_out, b_out):
    xp = jnp.zeros((_NP, _D), jnp.float32).at[:_N].set(x)
    loop = jnp.arange(_N, dtype=jnp.int32)
    src = jnp.concatenate([edge_index[0], loop])
    dst = jnp.concatenate([edge_index[1], loop])
    pad = _EPAD - src.shape[0]
    padv = jnp.full((pad,), _N, jnp.int32)
    src = jnp.concatenate([src, padv]).reshape(_NW, _NB, _B)
    dst = jnp.concatenate([dst, padv]).reshape(_NW, _NB, _B)

    M = jnp.repeat(jnp.eye(_H, dtype=jnp.float32), _HID, axis=0)
    am1 = jnp.concatenate(
        [M * a_src1.reshape(-1, 1), M * a_dst1.reshape(-1, 1)], axis=1)
    am2 = jnp.concatenate(
        [M * a_src2.reshape(-1, 1), M * a_dst2.reshape(-1, 1)], axis=1)
    expand = M.T
    wout = jnp.zeros((_D, _D), jnp.float32).at[:, :_C].set(W_out)
    bout = jnp.zeros((1, _D), jnp.float32).at[0, :_C].set(b_out)

    h1p, aa1, a1 = _mm_attn(xp, W1, am1)
    num1, den1 = _edge(h1p, aa1[:, :_H], aa1[:, _H:], a1, src, dst)

    h2p, aa2, a2 = _fin(num1, den1, expand, b1.reshape(1, -1), W2, am2)
    num2, den2 = _edge(h2p, aa2[:, :_H], aa2[:, _H:], a2, src, dst)

    outp = _out(num2, den2, expand, b2.reshape(1, -1), wout, bout)
    return outp[:_N, :_C]
